# baseline (speedup 1.0000x reference)
import functools

import jax
import jax.numpy as jnp
from jax import lax
from jax.experimental import pallas as pl
from jax.experimental.pallas import tpu as pltpu
from jax.experimental.pallas import tpu_sc as plsc

V = 1000001
D = 64
B = 4096
L = 200
N = B * L
NC = 2
NS = 16
NW = NC * NS
PER_W = N // NW
CH = 128
NCHUNK = PER_W // CH

BLKR = 2048
H = BLKR // 2
G1 = (V + BLKR - 1) // BLKR
V2 = G1 * BLKR


def _transform_body(x_ref, w1_ref, b1_ref, w2_ref, b2_ref, o_ref):
    x = x_ref[...]
    h = jnp.dot(x, w1_ref[...], preferred_element_type=jnp.float32) + b1_ref[...]
    y = jnp.dot(h, w2_ref[...], preferred_element_type=jnp.float32) + b2_ref[...]
    o_ref[...] = jnp.concatenate([y[:H], y[H:]], axis=1)


def _transform_table(table, W1, b1, W2, b2):
    return pl.pallas_call(
        _transform_body,
        grid=(G1,),
        in_specs=[
            pl.BlockSpec((BLKR, D), lambda i: (i, 0)),
            pl.BlockSpec((D, D), lambda i: (0, 0)),
            pl.BlockSpec((1, D), lambda i: (0, 0)),
            pl.BlockSpec((D, D), lambda i: (0, 0)),
            pl.BlockSpec((1, D), lambda i: (0, 0)),
        ],
        out_specs=pl.BlockSpec((H, 2 * D), lambda i: (i, 0)),
        out_shape=jax.ShapeDtypeStruct((V2 // 2, 2 * D), jnp.float32),
    )(table, W1, b1.reshape(1, D), W2, b2.reshape(1, D))


def _sc_gather(t_rows, idx_flat):
    mesh = plsc.VectorSubcoreMesh(core_axis_name="c", subcore_axis_name="s")

    @functools.partial(
        pl.kernel,
        out_type=jax.ShapeDtypeStruct((N, D), jnp.float32),
        mesh=mesh,
        scratch_types=[
            pltpu.VMEM((PER_W,), jnp.int32),
            pltpu.VMEM((CH, D), jnp.float32),
            pltpu.SemaphoreType.DMA,
        ],
        compiler_params=pltpu.CompilerParams(use_tc_tiling_on_sc=False),
    )
    def k(t_hbm, idx_hbm, out_hbm, idx_v, buf, sem):
        wid = lax.axis_index("s") * NC + lax.axis_index("c")
        base = wid * PER_W
        pltpu.sync_copy(idx_hbm.at[pl.ds(base, PER_W)], idx_v)

        def body(j, carry):
            pltpu.async_copy(t_hbm.at[idx_v.at[pl.ds(j * CH, CH)]], buf, sem).wait()
            pltpu.sync_copy(buf, out_hbm.at[pl.ds(base + j * CH, CH)])
            return carry

        lax.fori_loop(0, NCHUNK, body, 0)

    return k(t_rows, idx_flat)


def kernel(input_ids, emb_table, W1, b1, W2, b2):
    t_packed = _transform_table(emb_table, W1, b1, W2, b2)
    t_rows = t_packed.reshape(V2, D)
    v = input_ids.reshape(N)
    r = v % BLKR
    idx_flat = (v - r) + 2 * (r % H) + (r // H)
    y = _sc_gather(t_rows, idx_flat)
    return (y.reshape(B, L, D),)
